# baseline (speedup 1.0000x reference)
import jax
import jax.numpy as jnp
from jax.experimental import pallas as pl
from jax.experimental.pallas import tpu as pltpu

_MAX_BIN = 16
_RB = 1024
_TC = 256


def _dp_kernel(x_ref, o_ref, z_ref):
    t_total = x_ref.shape[3]
    tc = min(_TC, t_total)
    n_chunks = t_total // tc
    zeros = jnp.zeros((8, 128), jnp.float32)
    ones = jnp.ones((8, 128), jnp.float32)
    dp = (ones,) + (zeros,) * _MAX_BIN

    sub = jax.lax.broadcasted_iota(jnp.int32, (8, 128), 0)

    for c in range(n_chunks):
        par = c % 2
        for s in range(8):
            z_ref[par, s] = jnp.transpose(x_ref[0, s, :, pl.ds(c * tc, tc)])

        def body(i, dp):
            a = [z_ref[par, s, pl.ds(i * 8, 8)] for s in range(8)]
            for k in (1, 2, 4):
                b = list(a)
                for s in range(8):
                    shift = k if (s & k) == 0 else 8 - k
                    rolled = pltpu.roll(a[s ^ k], shift, axis=0)
                    b[s] = jnp.where(((sub ^ s) & k) == 0, a[s], rolled)
                a = b
            for j in range(8):
                p = a[j]
                q = 1.0 - p
                new = [dp[0] * q]
                for k in range(1, _MAX_BIN + 1):
                    new.append(dp[k] * q + dp[k - 1] * p)
                new[_MAX_BIN] = new[_MAX_BIN] + dp[_MAX_BIN]
                dp = tuple(new)
            return dp

        dp = jax.lax.fori_loop(0, tc // 8, body, dp)

    for k in range(_MAX_BIN + 1):
        o_ref[0, k] = dp[k]


def kernel(slice_probs) -> jnp.ndarray:
    B, T = slice_probs.shape
    nb = B // _RB
    x4 = slice_probs.reshape(nb, 8, 128, T)
    out = pl.pallas_call(
        _dp_kernel,
        grid=(nb,),
        in_specs=[pl.BlockSpec((1, 8, 128, T), lambda i: (i, 0, 0, 0))],
        out_specs=pl.BlockSpec((1, _MAX_BIN + 1, 8, 128), lambda i: (i, 0, 0, 0)),
        out_shape=jax.ShapeDtypeStruct((nb, _MAX_BIN + 1, 8, 128), jnp.float32),
        scratch_shapes=[pltpu.VMEM((2, 8, min(_TC, T), 128), jnp.float32)],
        compiler_params=pltpu.CompilerParams(
            dimension_semantics=("parallel",),
            vmem_limit_bytes=56 * 1024 * 1024,
        ),
        name="soft_count_dp",
    )(x4)
    return out.transpose(0, 2, 3, 1).reshape(B, _MAX_BIN + 1)


# speedup vs baseline: 1.0859x; 1.0859x over previous
import jax
import jax.numpy as jnp
from jax.experimental import pallas as pl
from jax.experimental.pallas import tpu as pltpu

_MAX_BIN = 16
_RB = 1024
_TC = 256
_UNROLL = 8


def _dp_kernel(x_ref, o_ref, z_ref, y_ref):
    t_total = x_ref.shape[3]
    tc = min(_TC, t_total)
    n_chunks = t_total // tc
    unroll = min(_UNROLL, tc)
    zeros = jnp.zeros((8, 128), jnp.float32)
    ones = jnp.ones((8, 128), jnp.float32)
    dp = (ones,) + (zeros,) * _MAX_BIN

    sub = jax.lax.broadcasted_iota(jnp.int32, (8, 128), 0)

    for c in range(n_chunks):
        for s in range(8):
            z_ref[s] = jnp.transpose(x_ref[0, s, :, pl.ds(c * tc, tc)])

        for g in range(tc // 8):
            a = [z_ref[s, pl.ds(8 * g, 8)] for s in range(8)]
            for k in (1, 2, 4):
                b = list(a)
                for s in range(8):
                    shift = k if (s & k) == 0 else 8 - k
                    rolled = pltpu.roll(a[s ^ k], shift, axis=0)
                    b[s] = jnp.where(((sub ^ s) & k) == 0, a[s], rolled)
                a = b
            for tp in range(8):
                y_ref[8 * g + tp] = a[tp]

        def body(i, dp):
            ps = y_ref[pl.ds(i * unroll, unroll)]
            for j in range(unroll):
                p = ps[j]
                q = 1.0 - p
                new = [dp[0] * q]
                for k in range(1, _MAX_BIN + 1):
                    new.append(dp[k] * q + dp[k - 1] * p)
                new[_MAX_BIN] = new[_MAX_BIN] + dp[_MAX_BIN]
                dp = tuple(new)
            return dp

        dp = jax.lax.fori_loop(0, tc // unroll, body, dp)

    for k in range(_MAX_BIN + 1):
        o_ref[0, k] = dp[k]


def kernel(slice_probs) -> jnp.ndarray:
    B, T = slice_probs.shape
    nb = B // _RB
    x4 = slice_probs.reshape(nb, 8, 128, T)
    tc = min(_TC, T)
    out = pl.pallas_call(
        _dp_kernel,
        grid=(nb,),
        in_specs=[pl.BlockSpec((1, 8, 128, T), lambda i: (i, 0, 0, 0))],
        out_specs=pl.BlockSpec((1, _MAX_BIN + 1, 8, 128), lambda i: (i, 0, 0, 0)),
        out_shape=jax.ShapeDtypeStruct((nb, _MAX_BIN + 1, 8, 128), jnp.float32),
        scratch_shapes=[pltpu.VMEM((8, tc, 128), jnp.float32),
                        pltpu.VMEM((tc, 8, 128), jnp.float32)],
        compiler_params=pltpu.CompilerParams(
            dimension_semantics=("parallel",),
            vmem_limit_bytes=56 * 1024 * 1024,
        ),
        name="soft_count_dp",
    )(x4)
    return out.transpose(0, 2, 3, 1).reshape(B, _MAX_BIN + 1)


# speedup vs baseline: 1.2127x; 1.1169x over previous
import jax
import jax.numpy as jnp
from jax.experimental import pallas as pl
from jax.experimental.pallas import tpu as pltpu

_MAX_BIN = 16
_RB = 1024
_TC = 256
_GROUPS_PER_ITER = 4


def _step4(dp, p):
    q = [1.0 - pj for pj in p]
    c20 = q[0] * q[1]
    c21 = q[0] * p[1] + p[0] * q[1]
    c22 = p[0] * p[1]
    c30 = c20 * q[2]
    c31 = c21 * q[2] + c20 * p[2]
    c32 = c22 * q[2] + c21 * p[2]
    c33 = c22 * p[2]
    a0 = c30 * q[3]
    a1 = c31 * q[3] + c30 * p[3]
    a2 = c32 * q[3] + c31 * p[3]
    a3 = c33 * q[3] + c32 * p[3]
    a4 = c33 * p[3]
    g = [2.0 - pj for pj in p]
    h2 = g[3] * g[2]
    h1 = h2 * g[1]
    g4 = h1 * g[0]
    u1 = h1 * p[0]
    u2 = h2 * p[1]
    u3 = g[3] * p[2]
    f0 = dp[15]
    f1 = q[0] * dp[15] + p[0] * dp[14]
    f2 = c20 * dp[15] + c21 * dp[14] + c22 * dp[13]
    f3 = c30 * dp[15] + c31 * dp[14] + c32 * dp[13] + c33 * dp[12]
    s16 = dp[16] * g4 + ((u1 * f0 + u2 * f1) + (u3 * f2 + p[3] * f3))
    a = (a0, a1, a2, a3, a4)
    new = []
    for k in range(_MAX_BIN):
        terms = [a[d] * dp[k - d] for d in range(min(k, 4) + 1)]
        while len(terms) > 1:
            terms = [terms[i] + terms[i + 1] for i in range(0, len(terms) - 1, 2)] \
                + ([terms[-1]] if len(terms) % 2 else [])
        new.append(terms[0])
    new.append(s16)
    return tuple(new)


def _dp_kernel(x_ref, o_ref, z_ref, y_ref):
    t_total = x_ref.shape[3]
    tc = min(_TC, t_total)
    n_chunks = t_total // tc
    span = 4 * _GROUPS_PER_ITER if tc % (4 * _GROUPS_PER_ITER) == 0 else 4
    zeros = jnp.zeros((8, 128), jnp.float32)
    ones = jnp.ones((8, 128), jnp.float32)
    dp = (ones,) + (zeros,) * _MAX_BIN

    sub = jax.lax.broadcasted_iota(jnp.int32, (8, 128), 0)

    for c in range(n_chunks):
        for s in range(8):
            z_ref[s] = jnp.transpose(x_ref[0, s, :, pl.ds(c * tc, tc)])

        for g in range(tc // 8):
            a = [z_ref[s, pl.ds(8 * g, 8)] for s in range(8)]
            for k in (1, 2, 4):
                b = list(a)
                for s in range(8):
                    shift = k if (s & k) == 0 else 8 - k
                    rolled = pltpu.roll(a[s ^ k], shift, axis=0)
                    b[s] = jnp.where(((sub ^ s) & k) == 0, a[s], rolled)
                a = b
            for tp in range(8):
                y_ref[8 * g + tp] = a[tp]

        def body(i, dp):
            ps = y_ref[pl.ds(i * span, span)]
            for j in range(span // 4):
                dp = _step4(dp, [ps[4 * j + d] for d in range(4)])
            return dp

        dp = jax.lax.fori_loop(0, tc // span, body, dp)

    for k in range(_MAX_BIN + 1):
        o_ref[0, k] = dp[k]


def kernel(slice_probs) -> jnp.ndarray:
    B, T = slice_probs.shape
    nb = B // _RB
    x4 = slice_probs.reshape(nb, 8, 128, T)
    tc = min(_TC, T)
    out = pl.pallas_call(
        _dp_kernel,
        grid=(nb,),
        in_specs=[pl.BlockSpec((1, 8, 128, T), lambda i: (i, 0, 0, 0))],
        out_specs=pl.BlockSpec((1, _MAX_BIN + 1, 8, 128), lambda i: (i, 0, 0, 0)),
        out_shape=jax.ShapeDtypeStruct((nb, _MAX_BIN + 1, 8, 128), jnp.float32),
        scratch_shapes=[pltpu.VMEM((8, tc, 128), jnp.float32),
                        pltpu.VMEM((tc, 8, 128), jnp.float32)],
        compiler_params=pltpu.CompilerParams(
            dimension_semantics=("parallel",),
            vmem_limit_bytes=56 * 1024 * 1024,
        ),
        name="soft_count_dp",
    )(x4)
    return out.transpose(0, 2, 3, 1).reshape(B, _MAX_BIN + 1)


# speedup vs baseline: 1.2478x; 1.0289x over previous
import jax
import jax.numpy as jnp
from jax.experimental import pallas as pl
from jax.experimental.pallas import tpu as pltpu

_MAX_BIN = 16
_RB = 1024
_TC = 256
_GROUPS_PER_ITER = 8


def _step4(dp, p):
    q = [1.0 - pj for pj in p]
    c20 = q[0] * q[1]
    c21 = q[0] * p[1] + p[0] * q[1]
    c22 = p[0] * p[1]
    c30 = c20 * q[2]
    c31 = c21 * q[2] + c20 * p[2]
    c32 = c22 * q[2] + c21 * p[2]
    c33 = c22 * p[2]
    a0 = c30 * q[3]
    a1 = c31 * q[3] + c30 * p[3]
    a2 = c32 * q[3] + c31 * p[3]
    a3 = c33 * q[3] + c32 * p[3]
    a4 = c33 * p[3]
    g = [2.0 - pj for pj in p]
    h2 = g[3] * g[2]
    h1 = h2 * g[1]
    g4 = h1 * g[0]
    u1 = h1 * p[0]
    u2 = h2 * p[1]
    u3 = g[3] * p[2]
    f0 = dp[15]
    f1 = q[0] * dp[15] + p[0] * dp[14]
    f2 = c20 * dp[15] + c21 * dp[14] + c22 * dp[13]
    f3 = c30 * dp[15] + c31 * dp[14] + c32 * dp[13] + c33 * dp[12]
    s16 = dp[16] * g4 + ((u1 * f0 + u2 * f1) + (u3 * f2 + p[3] * f3))
    a = (a0, a1, a2, a3, a4)
    new = []
    for k in range(_MAX_BIN):
        terms = [a[d] * dp[k - d] for d in range(min(k, 4) + 1)]
        while len(terms) > 1:
            terms = [terms[i] + terms[i + 1] for i in range(0, len(terms) - 1, 2)] \
                + ([terms[-1]] if len(terms) % 2 else [])
        new.append(terms[0])
    new.append(s16)
    return tuple(new)


def _dp_kernel(x_ref, o_ref, z_ref, y_ref):
    t_total = x_ref.shape[3]
    tc = min(_TC, t_total)
    n_chunks = t_total // tc
    span = 4 * _GROUPS_PER_ITER if tc % (4 * _GROUPS_PER_ITER) == 0 else 4
    zeros = jnp.zeros((8, 128), jnp.float32)
    ones = jnp.ones((8, 128), jnp.float32)
    dp = (ones,) + (zeros,) * _MAX_BIN

    sub = jax.lax.broadcasted_iota(jnp.int32, (8, 128), 0)

    for c in range(n_chunks):
        for s in range(8):
            z_ref[s] = jnp.transpose(x_ref[0, s, :, pl.ds(c * tc, tc)])

        for g in range(tc // 8):
            a = [z_ref[s, pl.ds(8 * g, 8)] for s in range(8)]
            for k in (1, 2, 4):
                b = list(a)
                for s in range(8):
                    shift = k if (s & k) == 0 else 8 - k
                    rolled = pltpu.roll(a[s ^ k], shift, axis=0)
                    b[s] = jnp.where(((sub ^ s) & k) == 0, a[s], rolled)
                a = b
            for tp in range(8):
                y_ref[8 * g + tp] = a[tp]

        def body(i, dp):
            ps = y_ref[pl.ds(i * span, span)]
            for j in range(span // 4):
                dp = _step4(dp, [ps[4 * j + d] for d in range(4)])
            return dp

        dp = jax.lax.fori_loop(0, tc // span, body, dp)

    for k in range(_MAX_BIN + 1):
        o_ref[0, k] = dp[k]


def kernel(slice_probs) -> jnp.ndarray:
    B, T = slice_probs.shape
    nb = B // _RB
    x4 = slice_probs.reshape(nb, 8, 128, T)
    tc = min(_TC, T)
    out = pl.pallas_call(
        _dp_kernel,
        grid=(nb,),
        in_specs=[pl.BlockSpec((1, 8, 128, T), lambda i: (i, 0, 0, 0))],
        out_specs=pl.BlockSpec((1, _MAX_BIN + 1, 8, 128), lambda i: (i, 0, 0, 0)),
        out_shape=jax.ShapeDtypeStruct((nb, _MAX_BIN + 1, 8, 128), jnp.float32),
        scratch_shapes=[pltpu.VMEM((8, tc, 128), jnp.float32),
                        pltpu.VMEM((tc, 8, 128), jnp.float32)],
        compiler_params=pltpu.CompilerParams(
            dimension_semantics=("parallel",),
            vmem_limit_bytes=56 * 1024 * 1024,
        ),
        name="soft_count_dp",
    )(x4)
    return out.transpose(0, 2, 3, 1).reshape(B, _MAX_BIN + 1)


# speedup vs baseline: 1.2678x; 1.0160x over previous
import jax
import jax.numpy as jnp
from jax.experimental import pallas as pl
from jax.experimental.pallas import tpu as pltpu

_MAX_BIN = 16
_RB = 1024
_TC = 256
_GROUPS_PER_ITER = 16


def _step4(dp, p):
    q = [1.0 - pj for pj in p]
    c20 = q[0] * q[1]
    c21 = q[0] * p[1] + p[0] * q[1]
    c22 = p[0] * p[1]
    c30 = c20 * q[2]
    c31 = c21 * q[2] + c20 * p[2]
    c32 = c22 * q[2] + c21 * p[2]
    c33 = c22 * p[2]
    a0 = c30 * q[3]
    a1 = c31 * q[3] + c30 * p[3]
    a2 = c32 * q[3] + c31 * p[3]
    a3 = c33 * q[3] + c32 * p[3]
    a4 = c33 * p[3]
    g = [2.0 - pj for pj in p]
    h2 = g[3] * g[2]
    h1 = h2 * g[1]
    g4 = h1 * g[0]
    u1 = h1 * p[0]
    u2 = h2 * p[1]
    u3 = g[3] * p[2]
    f0 = dp[15]
    f1 = q[0] * dp[15] + p[0] * dp[14]
    f2 = c20 * dp[15] + c21 * dp[14] + c22 * dp[13]
    f3 = c30 * dp[15] + c31 * dp[14] + c32 * dp[13] + c33 * dp[12]
    s16 = dp[16] * g4 + ((u1 * f0 + u2 * f1) + (u3 * f2 + p[3] * f3))
    a = (a0, a1, a2, a3, a4)
    new = []
    for k in range(_MAX_BIN):
        terms = [a[d] * dp[k - d] for d in range(min(k, 4) + 1)]
        while len(terms) > 1:
            terms = [terms[i] + terms[i + 1] for i in range(0, len(terms) - 1, 2)] \
                + ([terms[-1]] if len(terms) % 2 else [])
        new.append(terms[0])
    new.append(s16)
    return tuple(new)


def _dp_kernel(x_ref, o_ref, z_ref, y_ref):
    t_total = x_ref.shape[3]
    tc = min(_TC, t_total)
    n_chunks = t_total // tc
    span = 4 * _GROUPS_PER_ITER if tc % (4 * _GROUPS_PER_ITER) == 0 else 4
    zeros = jnp.zeros((8, 128), jnp.float32)
    ones = jnp.ones((8, 128), jnp.float32)
    dp = (ones,) + (zeros,) * _MAX_BIN

    sub = jax.lax.broadcasted_iota(jnp.int32, (8, 128), 0)

    for c in range(n_chunks):
        for s in range(8):
            z_ref[s] = jnp.transpose(x_ref[0, s, :, pl.ds(c * tc, tc)])

        for g in range(tc // 8):
            a = [z_ref[s, pl.ds(8 * g, 8)] for s in range(8)]
            for k in (1, 2, 4):
                b = list(a)
                for s in range(8):
                    shift = k if (s & k) == 0 else 8 - k
                    rolled = pltpu.roll(a[s ^ k], shift, axis=0)
                    b[s] = jnp.where(((sub ^ s) & k) == 0, a[s], rolled)
                a = b
            for tp in range(8):
                y_ref[8 * g + tp] = a[tp]

        def body(i, dp):
            ps = y_ref[pl.ds(i * span, span)]
            for j in range(span // 4):
                dp = _step4(dp, [ps[4 * j + d] for d in range(4)])
            return dp

        dp = jax.lax.fori_loop(0, tc // span, body, dp)

    for k in range(_MAX_BIN + 1):
        o_ref[0, k] = dp[k]


def kernel(slice_probs) -> jnp.ndarray:
    B, T = slice_probs.shape
    nb = B // _RB
    x4 = slice_probs.reshape(nb, 8, 128, T)
    tc = min(_TC, T)
    out = pl.pallas_call(
        _dp_kernel,
        grid=(nb,),
        in_specs=[pl.BlockSpec((1, 8, 128, T), lambda i: (i, 0, 0, 0))],
        out_specs=pl.BlockSpec((1, _MAX_BIN + 1, 8, 128), lambda i: (i, 0, 0, 0)),
        out_shape=jax.ShapeDtypeStruct((nb, _MAX_BIN + 1, 8, 128), jnp.float32),
        scratch_shapes=[pltpu.VMEM((8, tc, 128), jnp.float32),
                        pltpu.VMEM((tc, 8, 128), jnp.float32)],
        compiler_params=pltpu.CompilerParams(
            dimension_semantics=("parallel",),
            vmem_limit_bytes=56 * 1024 * 1024,
        ),
        name="soft_count_dp",
    )(x4)
    return out.transpose(0, 2, 3, 1).reshape(B, _MAX_BIN + 1)


# speedup vs baseline: 1.2768x; 1.0071x over previous
import jax
import jax.numpy as jnp
from jax.experimental import pallas as pl
from jax.experimental.pallas import tpu as pltpu

_MAX_BIN = 16
_RB = 1024
_TC = 256
_GROUPS_PER_ITER = 32


def _step4(dp, p):
    q = [1.0 - pj for pj in p]
    c20 = q[0] * q[1]
    c21 = q[0] * p[1] + p[0] * q[1]
    c22 = p[0] * p[1]
    c30 = c20 * q[2]
    c31 = c21 * q[2] + c20 * p[2]
    c32 = c22 * q[2] + c21 * p[2]
    c33 = c22 * p[2]
    a0 = c30 * q[3]
    a1 = c31 * q[3] + c30 * p[3]
    a2 = c32 * q[3] + c31 * p[3]
    a3 = c33 * q[3] + c32 * p[3]
    a4 = c33 * p[3]
    g = [2.0 - pj for pj in p]
    h2 = g[3] * g[2]
    h1 = h2 * g[1]
    g4 = h1 * g[0]
    u1 = h1 * p[0]
    u2 = h2 * p[1]
    u3 = g[3] * p[2]
    f0 = dp[15]
    f1 = q[0] * dp[15] + p[0] * dp[14]
    f2 = c20 * dp[15] + c21 * dp[14] + c22 * dp[13]
    f3 = c30 * dp[15] + c31 * dp[14] + c32 * dp[13] + c33 * dp[12]
    s16 = dp[16] * g4 + ((u1 * f0 + u2 * f1) + (u3 * f2 + p[3] * f3))
    a = (a0, a1, a2, a3, a4)
    new = []
    for k in range(_MAX_BIN):
        terms = [a[d] * dp[k - d] for d in range(min(k, 4) + 1)]
        while len(terms) > 1:
            terms = [terms[i] + terms[i + 1] for i in range(0, len(terms) - 1, 2)] \
                + ([terms[-1]] if len(terms) % 2 else [])
        new.append(terms[0])
    new.append(s16)
    return tuple(new)


def _dp_kernel(x_ref, o_ref, z_ref, y_ref):
    t_total = x_ref.shape[3]
    tc = min(_TC, t_total)
    n_chunks = t_total // tc
    span = 4 * _GROUPS_PER_ITER if tc % (4 * _GROUPS_PER_ITER) == 0 else 4
    zeros = jnp.zeros((8, 128), jnp.float32)
    ones = jnp.ones((8, 128), jnp.float32)
    dp = (ones,) + (zeros,) * _MAX_BIN

    sub = jax.lax.broadcasted_iota(jnp.int32, (8, 128), 0)

    for c in range(n_chunks):
        for s in range(8):
            z_ref[s] = jnp.transpose(x_ref[0, s, :, pl.ds(c * tc, tc)])

        for g in range(tc // 8):
            a = [z_ref[s, pl.ds(8 * g, 8)] for s in range(8)]
            for k in (1, 2, 4):
                b = list(a)
                for s in range(8):
                    shift = k if (s & k) == 0 else 8 - k
                    rolled = pltpu.roll(a[s ^ k], shift, axis=0)
                    b[s] = jnp.where(((sub ^ s) & k) == 0, a[s], rolled)
                a = b
            for tp in range(8):
                y_ref[8 * g + tp] = a[tp]

        def body(i, dp):
            ps = y_ref[pl.ds(i * span, span)]
            for j in range(span // 4):
                dp = _step4(dp, [ps[4 * j + d] for d in range(4)])
            return dp

        dp = jax.lax.fori_loop(0, tc // span, body, dp)

    for k in range(_MAX_BIN + 1):
        o_ref[0, k] = dp[k]


def kernel(slice_probs) -> jnp.ndarray:
    B, T = slice_probs.shape
    nb = B // _RB
    x4 = slice_probs.reshape(nb, 8, 128, T)
    tc = min(_TC, T)
    out = pl.pallas_call(
        _dp_kernel,
        grid=(nb,),
        in_specs=[pl.BlockSpec((1, 8, 128, T), lambda i: (i, 0, 0, 0))],
        out_specs=pl.BlockSpec((1, _MAX_BIN + 1, 8, 128), lambda i: (i, 0, 0, 0)),
        out_shape=jax.ShapeDtypeStruct((nb, _MAX_BIN + 1, 8, 128), jnp.float32),
        scratch_shapes=[pltpu.VMEM((8, tc, 128), jnp.float32),
                        pltpu.VMEM((tc, 8, 128), jnp.float32)],
        compiler_params=pltpu.CompilerParams(
            dimension_semantics=("parallel",),
            vmem_limit_bytes=56 * 1024 * 1024,
        ),
        name="soft_count_dp",
    )(x4)
    return out.transpose(0, 2, 3, 1).reshape(B, _MAX_BIN + 1)


# speedup vs baseline: 1.4260x; 1.1169x over previous
import jax
import jax.numpy as jnp
from jax.experimental import pallas as pl
from jax.experimental.pallas import tpu as pltpu

_MAX_BIN = 16
_RB = 1024
_TC = 256
_GROUPS_PER_ITER = 64


def _step4(dp, p):
    q = [1.0 - pj for pj in p]
    c20 = q[0] * q[1]
    c21 = q[0] * p[1] + p[0] * q[1]
    c22 = p[0] * p[1]
    c30 = c20 * q[2]
    c31 = c21 * q[2] + c20 * p[2]
    c32 = c22 * q[2] + c21 * p[2]
    c33 = c22 * p[2]
    a0 = c30 * q[3]
    a1 = c31 * q[3] + c30 * p[3]
    a2 = c32 * q[3] + c31 * p[3]
    a3 = c33 * q[3] + c32 * p[3]
    a4 = c33 * p[3]
    g = [2.0 - pj for pj in p]
    h2 = g[3] * g[2]
    h1 = h2 * g[1]
    g4 = h1 * g[0]
    u1 = h1 * p[0]
    u2 = h2 * p[1]
    u3 = g[3] * p[2]
    f0 = dp[15]
    f1 = q[0] * dp[15] + p[0] * dp[14]
    f2 = c20 * dp[15] + c21 * dp[14] + c22 * dp[13]
    f3 = c30 * dp[15] + c31 * dp[14] + c32 * dp[13] + c33 * dp[12]
    s16 = dp[16] * g4 + ((u1 * f0 + u2 * f1) + (u3 * f2 + p[3] * f3))
    a = (a0, a1, a2, a3, a4)
    new = []
    for k in range(_MAX_BIN):
        terms = [a[d] * dp[k - d] for d in range(min(k, 4) + 1)]
        while len(terms) > 1:
            terms = [terms[i] + terms[i + 1] for i in range(0, len(terms) - 1, 2)] \
                + ([terms[-1]] if len(terms) % 2 else [])
        new.append(terms[0])
    new.append(s16)
    return tuple(new)


def _dp_kernel(x_ref, o_ref, z_ref, y_ref):
    t_total = x_ref.shape[3]
    tc = min(_TC, t_total)
    n_chunks = t_total // tc
    span = 4 * _GROUPS_PER_ITER if tc % (4 * _GROUPS_PER_ITER) == 0 else 4
    zeros = jnp.zeros((8, 128), jnp.float32)
    ones = jnp.ones((8, 128), jnp.float32)
    dp = (ones,) + (zeros,) * _MAX_BIN

    sub = jax.lax.broadcasted_iota(jnp.int32, (8, 128), 0)

    for c in range(n_chunks):
        for s in range(8):
            z_ref[s] = jnp.transpose(x_ref[0, s, :, pl.ds(c * tc, tc)])

        for g in range(tc // 8):
            a = [z_ref[s, pl.ds(8 * g, 8)] for s in range(8)]
            for k in (1, 2, 4):
                b = list(a)
                for s in range(8):
                    shift = k if (s & k) == 0 else 8 - k
                    rolled = pltpu.roll(a[s ^ k], shift, axis=0)
                    b[s] = jnp.where(((sub ^ s) & k) == 0, a[s], rolled)
                a = b
            for tp in range(8):
                y_ref[8 * g + tp] = a[tp]

        def body(i, dp):
            ps = y_ref[pl.ds(i * span, span)]
            for j in range(span // 4):
                dp = _step4(dp, [ps[4 * j + d] for d in range(4)])
            return dp

        dp = jax.lax.fori_loop(0, tc // span, body, dp)

    for k in range(_MAX_BIN + 1):
        o_ref[0, k] = dp[k]


def kernel(slice_probs) -> jnp.ndarray:
    B, T = slice_probs.shape
    nb = B // _RB
    x4 = slice_probs.reshape(nb, 8, 128, T)
    tc = min(_TC, T)
    out = pl.pallas_call(
        _dp_kernel,
        grid=(nb,),
        in_specs=[pl.BlockSpec((1, 8, 128, T), lambda i: (i, 0, 0, 0))],
        out_specs=pl.BlockSpec((1, _MAX_BIN + 1, 8, 128), lambda i: (i, 0, 0, 0)),
        out_shape=jax.ShapeDtypeStruct((nb, _MAX_BIN + 1, 8, 128), jnp.float32),
        scratch_shapes=[pltpu.VMEM((8, tc, 128), jnp.float32),
                        pltpu.VMEM((tc, 8, 128), jnp.float32)],
        compiler_params=pltpu.CompilerParams(
            dimension_semantics=("parallel",),
            vmem_limit_bytes=56 * 1024 * 1024,
        ),
        name="soft_count_dp",
    )(x4)
    return out.transpose(0, 2, 3, 1).reshape(B, _MAX_BIN + 1)


# speedup vs baseline: 1.4262x; 1.0002x over previous
import jax
import jax.numpy as jnp
from jax.experimental import pallas as pl
from jax.experimental.pallas import tpu as pltpu

_MAX_BIN = 16
_RB = 1024
_TC = 256
_GROUPS_PER_ITER = 64


def _step4(dp, p):
    q = [1.0 - pj for pj in p]
    c20 = q[0] * q[1]
    c21 = q[0] * p[1] + p[0] * q[1]
    c22 = p[0] * p[1]
    c30 = c20 * q[2]
    c31 = c21 * q[2] + c20 * p[2]
    c32 = c22 * q[2] + c21 * p[2]
    c33 = c22 * p[2]
    a0 = c30 * q[3]
    a1 = c31 * q[3] + c30 * p[3]
    a2 = c32 * q[3] + c31 * p[3]
    a3 = c33 * q[3] + c32 * p[3]
    a4 = c33 * p[3]
    g = [2.0 - pj for pj in p]
    h2 = g[3] * g[2]
    h1 = h2 * g[1]
    g4 = h1 * g[0]
    u1 = h1 * p[0]
    u2 = h2 * p[1]
    u3 = g[3] * p[2]
    f0 = dp[15]
    f1 = q[0] * dp[15] + p[0] * dp[14]
    f2 = c20 * dp[15] + c21 * dp[14] + c22 * dp[13]
    f3 = c30 * dp[15] + c31 * dp[14] + c32 * dp[13] + c33 * dp[12]
    s16 = dp[16] * g4 + ((u1 * f0 + u2 * f1) + (u3 * f2 + p[3] * f3))
    a = (a0, a1, a2, a3, a4)
    new = []
    for k in range(_MAX_BIN):
        terms = [a[d] * dp[k - d] for d in range(min(k, 4) + 1)]
        while len(terms) > 1:
            terms = [terms[i] + terms[i + 1] for i in range(0, len(terms) - 1, 2)] \
                + ([terms[-1]] if len(terms) % 2 else [])
        new.append(terms[0])
    new.append(s16)
    return tuple(new)


def _dp_kernel(x_ref, o_ref, z_ref):
    t_total = x_ref.shape[3]
    tc = min(_TC, t_total)
    n_chunks = t_total // tc
    zeros = jnp.zeros((8, 128), jnp.float32)
    ones = jnp.ones((8, 128), jnp.float32)
    dp = (ones,) + (zeros,) * _MAX_BIN

    sub = jax.lax.broadcasted_iota(jnp.int32, (8, 128), 0)

    for c in range(n_chunks):
        for s in range(8):
            z_ref[s] = jnp.transpose(x_ref[0, s, :, pl.ds(c * tc, tc)])

        for g in range(tc // 8):
            a = [z_ref[s, pl.ds(8 * g, 8)] for s in range(8)]
            for k in (1, 2, 4):
                b = list(a)
                for s in range(8):
                    shift = k if (s & k) == 0 else 8 - k
                    rolled = pltpu.roll(a[s ^ k], shift, axis=0)
                    b[s] = jnp.where(((sub ^ s) & k) == 0, a[s], rolled)
                a = b
            dp = _step4(dp, a[0:4])
            dp = _step4(dp, a[4:8])

    for k in range(_MAX_BIN + 1):
        o_ref[0, k] = dp[k]


def kernel(slice_probs) -> jnp.ndarray:
    B, T = slice_probs.shape
    nb = B // _RB
    x4 = slice_probs.reshape(nb, 8, 128, T)
    tc = min(_TC, T)
    out = pl.pallas_call(
        _dp_kernel,
        grid=(nb,),
        in_specs=[pl.BlockSpec((1, 8, 128, T), lambda i: (i, 0, 0, 0))],
        out_specs=pl.BlockSpec((1, _MAX_BIN + 1, 8, 128), lambda i: (i, 0, 0, 0)),
        out_shape=jax.ShapeDtypeStruct((nb, _MAX_BIN + 1, 8, 128), jnp.float32),
        scratch_shapes=[pltpu.VMEM((8, tc, 128), jnp.float32)],
        compiler_params=pltpu.CompilerParams(
            dimension_semantics=("parallel",),
            vmem_limit_bytes=56 * 1024 * 1024,
        ),
        name="soft_count_dp",
    )(x4)
    return out.transpose(0, 2, 3, 1).reshape(B, _MAX_BIN + 1)


# speedup vs baseline: 1.4264x; 1.0001x over previous
import jax
import jax.numpy as jnp
from jax.experimental import pallas as pl
from jax.experimental.pallas import tpu as pltpu

_MAX_BIN = 16
_RB = 1024
_TC = 256
_GROUPS_PER_ITER = 64


def _step4(dp, p):
    q = [1.0 - pj for pj in p]
    c20 = q[0] * q[1]
    c21 = q[0] * p[1] + p[0] * q[1]
    c22 = p[0] * p[1]
    c30 = c20 * q[2]
    c31 = c21 * q[2] + c20 * p[2]
    c32 = c22 * q[2] + c21 * p[2]
    c33 = c22 * p[2]
    a0 = c30 * q[3]
    a1 = c31 * q[3] + c30 * p[3]
    a2 = c32 * q[3] + c31 * p[3]
    a3 = c33 * q[3] + c32 * p[3]
    a4 = c33 * p[3]
    g = [2.0 - pj for pj in p]
    h2 = g[3] * g[2]
    h1 = h2 * g[1]
    g4 = h1 * g[0]
    u1 = h1 * p[0]
    u2 = h2 * p[1]
    u3 = g[3] * p[2]
    f0 = dp[15]
    f1 = q[0] * dp[15] + p[0] * dp[14]
    f2 = c20 * dp[15] + c21 * dp[14] + c22 * dp[13]
    f3 = c30 * dp[15] + c31 * dp[14] + c32 * dp[13] + c33 * dp[12]
    s16 = dp[16] * g4 + ((u1 * f0 + u2 * f1) + (u3 * f2 + p[3] * f3))
    a = (a0, a1, a2, a3, a4)
    new = []
    for k in range(_MAX_BIN):
        terms = [a[d] * dp[k - d] for d in range(min(k, 4) + 1)]
        while len(terms) > 1:
            terms = [terms[i] + terms[i + 1] for i in range(0, len(terms) - 1, 2)] \
                + ([terms[-1]] if len(terms) % 2 else [])
        new.append(terms[0])
    new.append(s16)
    return tuple(new)


def _dp_kernel(x_ref, o_ref, z_ref):
    t_total = x_ref.shape[3]
    tc = min(_TC, t_total)
    n_chunks = t_total // tc
    zeros = jnp.zeros((8, 128), jnp.float32)
    ones = jnp.ones((8, 128), jnp.float32)
    dp = (ones,) + (zeros,) * _MAX_BIN

    sub = jax.lax.broadcasted_iota(jnp.int32, (8, 128), 0)

    for c in range(n_chunks):
        par = c % 2
        for s in range(8):
            z_ref[par, s] = jnp.transpose(x_ref[0, s, :, pl.ds(c * tc, tc)])

        for g in range(tc // 8):
            a = [z_ref[par, s, pl.ds(8 * g, 8)] for s in range(8)]
            for k in (1, 2, 4):
                b = list(a)
                for s in range(8):
                    shift = k if (s & k) == 0 else 8 - k
                    rolled = pltpu.roll(a[s ^ k], shift, axis=0)
                    b[s] = jnp.where(((sub ^ s) & k) == 0, a[s], rolled)
                a = b
            dp = _step4(dp, a[0:4])
            dp = _step4(dp, a[4:8])

    for k in range(_MAX_BIN + 1):
        o_ref[0, k] = dp[k]


def kernel(slice_probs) -> jnp.ndarray:
    B, T = slice_probs.shape
    nb = B // _RB
    x4 = slice_probs.reshape(nb, 8, 128, T)
    tc = min(_TC, T)
    out = pl.pallas_call(
        _dp_kernel,
        grid=(nb,),
        in_specs=[pl.BlockSpec((1, 8, 128, T), lambda i: (i, 0, 0, 0))],
        out_specs=pl.BlockSpec((1, _MAX_BIN + 1, 8, 128), lambda i: (i, 0, 0, 0)),
        out_shape=jax.ShapeDtypeStruct((nb, _MAX_BIN + 1, 8, 128), jnp.float32),
        scratch_shapes=[pltpu.VMEM((2, 8, tc, 128), jnp.float32)],
        compiler_params=pltpu.CompilerParams(
            dimension_semantics=("parallel",),
            vmem_limit_bytes=56 * 1024 * 1024,
        ),
        name="soft_count_dp",
    )(x4)
    return out.transpose(0, 2, 3, 1).reshape(B, _MAX_BIN + 1)
